# baseline (speedup 1.0000x reference)
import functools

import jax
import jax.numpy as jnp
from jax import lax
from jax.experimental import pallas as pl
from jax.experimental.pallas import tpu as pltpu
from jax.experimental.pallas import tpu_sc as plsc

MU2 = 7.0
D = 64
B = 16384
L = 16
NC = 2
NS = 16
NW = NC * NS
BW = B // NW
NG = BW // L


def _mf_kernel(u_idx_hbm, i_idx_hbm, u_emb_hbm, i_emb_hbm, u_bias_hbm,
               i_bias_hbm, out_hbm, u_idx_v, i_idx_v, u_rows, i_rows,
               u_b_v, i_b_v, out_v, sem):
  wid = lax.axis_index("s") * NC + lax.axis_index("c")
  base = wid * BW

  pltpu.sync_copy(u_idx_hbm.at[pl.ds(base, BW)], u_idx_v)
  pltpu.sync_copy(i_idx_hbm.at[pl.ds(base, BW)], i_idx_v)

  c0 = pltpu.async_copy(u_emb_hbm.at[u_idx_v], u_rows, sem)
  c1 = pltpu.async_copy(i_emb_hbm.at[i_idx_v], i_rows, sem)
  c2 = pltpu.async_copy(u_bias_hbm.at[u_idx_v], u_b_v, sem)
  c3 = pltpu.async_copy(i_bias_hbm.at[i_idx_v], i_b_v, sem)
  c0.wait()
  c1.wait()
  c2.wait()
  c3.wait()

  def body(g, carry):
    gbase = g * L
    rows16 = gbase + lax.iota(jnp.int32, L)
    col = jnp.zeros((L,), jnp.int32)
    acc0 = u_b_v[pl.ds(gbase, L)] + i_b_v[pl.ds(gbase, L)] + MU2
    acc1 = jnp.zeros((L,), jnp.float32)
    acc2 = jnp.zeros((L,), jnp.float32)
    acc3 = jnp.zeros((L,), jnp.float32)
    accs = [acc0, acc1, acc2, acc3]
    for jd in range(D):
      ug = plsc.load_gather(u_rows, [rows16, col])
      vg = plsc.load_gather(i_rows, [rows16, col])
      accs[jd % 4] = accs[jd % 4] + ug * vg
      col = col + 1
    out_v[pl.ds(gbase, L)] = (accs[0] + accs[1]) + (accs[2] + accs[3])
    return carry

  lax.fori_loop(0, NG, body, 0)
  pltpu.sync_copy(out_v, out_hbm.at[pl.ds(base, BW)])


@jax.jit
def _mf(user_indices, item_indices, user_embedding, item_embedding,
        user_bias_flat, item_bias_flat):
  mesh = plsc.VectorSubcoreMesh(core_axis_name="c", subcore_axis_name="s")
  return pl.kernel(
      _mf_kernel,
      out_type=jax.ShapeDtypeStruct((B,), jnp.float32),
      mesh=mesh,
      scratch_types=[
          pltpu.VMEM((BW,), jnp.int32),
          pltpu.VMEM((BW,), jnp.int32),
          pltpu.VMEM((BW, D), jnp.float32),
          pltpu.VMEM((BW, D), jnp.float32),
          pltpu.VMEM((BW,), jnp.float32),
          pltpu.VMEM((BW,), jnp.float32),
          pltpu.VMEM((BW,), jnp.float32),
          pltpu.SemaphoreType.DMA,
      ],
      compiler_params=pltpu.CompilerParams(needs_layout_passes=False,
                                           use_tc_tiling_on_sc=False),
  )(user_indices, item_indices, user_embedding, item_embedding,
    user_bias_flat, item_bias_flat)


def kernel(user_indices, item_indices, user_embedding, item_embedding,
           user_bias, item_bias):
  return _mf(user_indices.astype(jnp.int32), item_indices.astype(jnp.int32),
             user_embedding, item_embedding, user_bias.reshape(-1),
             item_bias.reshape(-1))
